# Initial kernel scaffold; baseline (speedup 1.0000x reference)
import jax
import jax.numpy as jnp
from jax.experimental import pallas as pl


def kernel(feature, edge_index, edge_weight, W_s, b_s, W_i, b_i, W_r, b_r, bn_gamma, bn_beta, toI_W, toI_b, toR_W, toR_b, out_W, out_b):
    raise NotImplementedError("write your pallas kernel here")


# speedup vs baseline: 5.8664x; 5.8664x over previous
import functools

import jax
import jax.numpy as jnp
from jax import lax
from jax.experimental import pallas as pl
from jax.experimental.pallas import tpu as pltpu
from jax.experimental.pallas import tpu_sc as plsc

_EPS = 1e-5


def _k1_body(f_ref, ws_ref, wi_ref, wr_ref, bs_ref, bi_ref, br_ref,
             zs_ref, zi_ref, zr_ref, stats_ref):
    @pl.when(pl.program_id(0) == 0)
    def _():
        stats_ref[...] = jnp.zeros_like(stats_ref)

    f = f_ref[...]
    dn = (((1,), (1,)), ((), ()))
    zs = lax.dot_general(f, ws_ref[...], dn, preferred_element_type=jnp.float32) + f + bs_ref[...]
    zi = lax.dot_general(f, wi_ref[...], dn, preferred_element_type=jnp.float32) + f + bi_ref[...]
    zr = lax.dot_general(f, wr_ref[...], dn, preferred_element_type=jnp.float32) + f + br_ref[...]
    zs_ref[...] = zs
    zi_ref[...] = zi
    zr_ref[...] = zr
    upd = jnp.concatenate(
        [jnp.sum(zs, 0, keepdims=True), jnp.sum(zs * zs, 0, keepdims=True),
         jnp.sum(zi, 0, keepdims=True), jnp.sum(zi * zi, 0, keepdims=True),
         jnp.sum(zr, 0, keepdims=True), jnp.sum(zr * zr, 0, keepdims=True),
         jnp.zeros((2, zs.shape[1]), jnp.float32)], axis=0)
    stats_ref[...] += upd


def _bn_coefs(stats, row, gamma, beta, n):
    mean = stats[row:row + 1, :] * (1.0 / n)
    var = stats[row + 1:row + 2, :] * (1.0 / n) - mean * mean
    sc = gamma * lax.rsqrt(var + _EPS)
    sh = beta - mean * sc
    return sc, sh


def _k2_body(n, zi_ref, stats_ref, g_ref, b_ref, i_ref):
    sc, sh = _bn_coefs(stats_ref[...], 2, g_ref[...], b_ref[...], n)
    i_ref[...] = jnp.maximum(zi_ref[...] * sc + sh, 0.0)


def _k3_body(n, zs_ref, zr_ref, i_ref, n0_ref, n1_ref, stats_ref, g_ref, b_ref,
             a_ref, bm_ref, rw_ref, o1_ref, o2_ref, o3_ref,
             tib_ref, trb_ref, ob_ref, out_ref):
    st = stats_ref[...]
    g = g_ref[...]
    be = b_ref[...]
    sc_s, sh_s = _bn_coefs(st, 0, g, be, n)
    sc_r, sh_r = _bn_coefs(st, 4, g, be, n)
    s = jnp.maximum(zs_ref[...] * sc_s + sh_s, 0.0)
    r = jnp.maximum(zr_ref[...] * sc_r + sh_r, 0.0)
    i = i_ref[...]
    nb = n0_ref[...] + n1_ref[...]

    o1 = o1_ref[...]
    o2 = o2_ref[...]
    o3 = o3_ref[...]
    p21 = o2 - o1
    p32 = o3 - o2
    dnm = (((1,), (0,)), ((), ()))
    dnt = (((1,), (1,)), ((), ()))
    g_s = o1 + lax.dot_general(p21, a_ref[...], dnm, preferred_element_type=jnp.float32)
    g_i = o2 + lax.dot_general(p32, rw_ref[...], dnm, preferred_element_type=jnp.float32)
    g_n = lax.dot_general(p21, bm_ref[...], dnm, preferred_element_type=jnp.float32)
    c = (lax.dot_general(tib_ref[...], p21, dnt, preferred_element_type=jnp.float32)
         + lax.dot_general(trb_ref[...], p32, dnt, preferred_element_type=jnp.float32)
         + ob_ref[...])

    x = (lax.dot_general(s, g_s, dnt, preferred_element_type=jnp.float32)
         + lax.dot_general(i, g_i, dnt, preferred_element_type=jnp.float32)
         + lax.dot_general(r, o3, dnt, preferred_element_type=jnp.float32)
         + lax.dot_general(nb, g_n, dnt, preferred_element_type=jnp.float32)
         + c)
    x = x - jnp.max(x, axis=-1, keepdims=True)
    e = jnp.exp(x)
    out_ref[...] = e / jnp.sum(e, axis=-1, keepdims=True)


@functools.cache
def _make_sc_scatter(n, d, e, nc, ns, c):
    nw = nc * ns
    epw = e // nw
    nchunk = epw // c
    npc = 25
    nstage = nchunk // npc
    rpt = (n // ns) // 8 * 8
    tail = n - rpt * ns
    mesh = plsc.VectorSubcoreMesh(core_axis_name="c", subcore_axis_name="s")

    @functools.partial(
        pl.kernel,
        out_type=jax.ShapeDtypeStruct((nc, n, d), jnp.float32),
        mesh=mesh,
        scratch_types=[
            pltpu.VMEM((npc, c), jnp.int32),
            pltpu.VMEM((npc, c), jnp.int32),
            pltpu.VMEM((npc, c), jnp.float32),
            pltpu.VMEM((c, d), jnp.float32),
            pltpu.VMEM_SHARED((n, d), jnp.float32),
            pltpu.SemaphoreType.DMA,
        ],
    )
    def sc_scatter(i_hbm, src_hbm, dst_hbm, w_hbm, z_hbm, out_hbm,
                   src_v, dst_v, w_v, rows_v, acc, sem):
        ci = lax.axis_index("c")
        si = lax.axis_index("s")
        wid = si * nc + ci
        r0 = pl.multiple_of(si * rpt, 8)
        pltpu.sync_copy(z_hbm.at[pl.ds(r0, rpt)], acc.at[pl.ds(r0, rpt)])
        if tail:
            @pl.when(si == ns - 1)
            def _():
                pltpu.sync_copy(z_hbm.at[pl.ds(rpt * ns, tail)],
                                acc.at[pl.ds(rpt * ns, tail)])
        plsc.subcore_barrier()

        def stage(st, carry):
            pltpu.sync_copy(src_hbm.at[wid, st], src_v)
            pltpu.sync_copy(dst_hbm.at[wid, st], dst_v)
            pltpu.sync_copy(w_hbm.at[wid, st], w_v)

            def chunk(j, c2):
                pltpu.async_copy(i_hbm.at[src_v.at[j]], rows_v, sem).wait()
                for g in range(c // 16):
                    wv = w_v[j, pl.ds(g * 16, 16)]
                    for e in range(16):
                        w = wv[e]
                        ei = g * 16 + e
                        for k in range(d // 16):
                            sl = pl.ds(k * 16, 16)
                            rows_v[ei, sl] = rows_v[ei, sl] * w
                pltpu.sync_copy(rows_v, acc.at[dst_v.at[j]], add=True)
                return c2

            lax.fori_loop(0, npc, chunk, 0)
            return carry

        lax.fori_loop(0, nstage, stage, 0)
        plsc.subcore_barrier()
        pltpu.sync_copy(acc.at[pl.ds(r0, rpt)], out_hbm.at[ci, pl.ds(r0, rpt)])
        if tail:
            @pl.when(si == ns - 1)
            def _():
                pltpu.sync_copy(acc.at[pl.ds(rpt * ns, tail)],
                                out_hbm.at[ci, pl.ds(rpt * ns, tail)])

    return sc_scatter


def _sc_partials(i_arr, src3, dst3, w3, zeros, nc, ns, c):
    n, d = i_arr.shape
    e = src3.size
    return _make_sc_scatter(n, d, e, nc, ns, c)(i_arr, src3, dst3, w3, zeros)


def kernel(feature, edge_index, edge_weight, W_s, b_s, W_i, b_i, W_r, b_r,
           bn_gamma, bn_beta, toI_W, toI_b, toR_W, toR_b, out_W, out_b):
    n, d = feature.shape
    e = edge_weight.shape[0]
    blk = 1000 if n % 1000 == 0 else n
    nblk = n // blk
    nf = float(n)

    bs = b_s.reshape(1, d)
    bi = b_i.reshape(1, d)
    br = b_r.reshape(1, d)
    gam = bn_gamma.reshape(1, d)
    bet = bn_beta.reshape(1, d)

    row = lambda i: pl.BlockSpec((blk, d), lambda b: (b, 0))
    full = lambda s: pl.BlockSpec(s, lambda b: (0,) * len(s))

    z_s, z_i, z_r, stats = pl.pallas_call(
        _k1_body,
        grid=(nblk,),
        in_specs=[row(0)] + [full((d, d))] * 3 + [full((1, d))] * 3,
        out_specs=[row(0), row(0), row(0), full((8, d))],
        out_shape=[jax.ShapeDtypeStruct((n, d), jnp.float32)] * 3
        + [jax.ShapeDtypeStruct((8, d), jnp.float32)],
    )(feature, W_s, W_i, W_r, bs, bi, br)

    i_arr = pl.pallas_call(
        functools.partial(_k2_body, nf),
        grid=(nblk,),
        in_specs=[row(0), full((8, d)), full((1, d)), full((1, d))],
        out_specs=row(0),
        out_shape=jax.ShapeDtypeStruct((n, d), jnp.float32),
    )(z_i, stats, gam, bet)

    info = plsc.get_sparse_core_info()
    nc, ns = info.num_cores, info.num_subcores
    c = 80
    npc = 25
    src3 = edge_index[0].reshape(nc * ns, -1, npc, c)
    dst3 = edge_index[1].reshape(nc * ns, -1, npc, c)
    w3 = edge_weight.reshape(nc * ns, -1, npc, c)
    zeros = jnp.zeros((n, d), jnp.float32)
    partials = _sc_partials(i_arr, src3, dst3, w3, zeros, nc, ns, c)

    a_m = toI_W[:, :d]
    b_m = toI_W[:, d:]
    o1 = out_W[:, :d]
    o2 = out_W[:, d:2 * d]
    o3 = out_W[:, 2 * d:]

    out = pl.pallas_call(
        functools.partial(_k3_body, nf),
        grid=(nblk,),
        in_specs=[row(0)] * 5 + [full((8, d)), full((1, d)), full((1, d))]
        + [full((d, d))] * 3 + [full((3, d))] * 3
        + [full((1, d)), full((1, d)), full((1, 3))],
        out_specs=pl.BlockSpec((blk, 3), lambda b: (b, 0)),
        out_shape=jax.ShapeDtypeStruct((n, 3), jnp.float32),
    )(z_s, z_r, i_arr, partials[0], partials[1], stats, gam, bet,
      a_m, b_m, toR_W, o1, o2, o3,
      toI_b.reshape(1, d), toR_b.reshape(1, d), out_b.reshape(1, 3))
    return out


# speedup vs baseline: 8.0066x; 1.3648x over previous
import functools

import jax
import jax.numpy as jnp
from jax import lax
from jax.experimental import pallas as pl
from jax.experimental.pallas import tpu as pltpu
from jax.experimental.pallas import tpu_sc as plsc

_EPS = 1e-5


def _k1_body(f_ref, ws_ref, wi_ref, wr_ref, bs_ref, bi_ref, br_ref,
             zs_ref, zi_ref, zr_ref, stats_ref):
    @pl.when(pl.program_id(0) == 0)
    def _():
        stats_ref[...] = jnp.zeros_like(stats_ref)

    f = f_ref[...]
    dn = (((1,), (1,)), ((), ()))
    zs = lax.dot_general(f, ws_ref[...], dn, preferred_element_type=jnp.float32) + f + bs_ref[...]
    zi = lax.dot_general(f, wi_ref[...], dn, preferred_element_type=jnp.float32) + f + bi_ref[...]
    zr = lax.dot_general(f, wr_ref[...], dn, preferred_element_type=jnp.float32) + f + br_ref[...]
    zs_ref[...] = zs
    zi_ref[...] = zi
    zr_ref[...] = zr
    upd = jnp.concatenate(
        [jnp.sum(zs, 0, keepdims=True), jnp.sum(zs * zs, 0, keepdims=True),
         jnp.sum(zi, 0, keepdims=True), jnp.sum(zi * zi, 0, keepdims=True),
         jnp.sum(zr, 0, keepdims=True), jnp.sum(zr * zr, 0, keepdims=True),
         jnp.zeros((2, zs.shape[1]), jnp.float32)], axis=0)
    stats_ref[...] += upd


def _bn_coefs(stats, row, gamma, beta, n):
    mean = stats[row:row + 1, :] * (1.0 / n)
    var = stats[row + 1:row + 2, :] * (1.0 / n) - mean * mean
    sc = gamma * lax.rsqrt(var + _EPS)
    sh = beta - mean * sc
    return sc, sh


def _k2_body(n, zi_ref, stats_ref, g_ref, b_ref, i_ref):
    sc, sh = _bn_coefs(stats_ref[...], 2, g_ref[...], b_ref[...], n)
    i_ref[...] = jnp.maximum(zi_ref[...] * sc + sh, 0.0)


def _k3_body(n, zs_ref, zr_ref, i_ref, n0_ref, n1_ref, stats_ref, g_ref, b_ref,
             a_ref, bm_ref, rw_ref, o1_ref, o2_ref, o3_ref,
             tib_ref, trb_ref, ob_ref, out_ref):
    st = stats_ref[...]
    g = g_ref[...]
    be = b_ref[...]
    sc_s, sh_s = _bn_coefs(st, 0, g, be, n)
    sc_r, sh_r = _bn_coefs(st, 4, g, be, n)
    s = jnp.maximum(zs_ref[...] * sc_s + sh_s, 0.0)
    r = jnp.maximum(zr_ref[...] * sc_r + sh_r, 0.0)
    i = i_ref[...]
    nb = n0_ref[...] + n1_ref[...]

    o1 = o1_ref[...]
    o2 = o2_ref[...]
    o3 = o3_ref[...]
    p21 = o2 - o1
    p32 = o3 - o2
    dnm = (((1,), (0,)), ((), ()))
    dnt = (((1,), (1,)), ((), ()))
    g_s = o1 + lax.dot_general(p21, a_ref[...], dnm, preferred_element_type=jnp.float32)
    g_i = o2 + lax.dot_general(p32, rw_ref[...], dnm, preferred_element_type=jnp.float32)
    g_n = lax.dot_general(p21, bm_ref[...], dnm, preferred_element_type=jnp.float32)
    c = (lax.dot_general(tib_ref[...], p21, dnt, preferred_element_type=jnp.float32)
         + lax.dot_general(trb_ref[...], p32, dnt, preferred_element_type=jnp.float32)
         + ob_ref[...])

    x = (lax.dot_general(s, g_s, dnt, preferred_element_type=jnp.float32)
         + lax.dot_general(i, g_i, dnt, preferred_element_type=jnp.float32)
         + lax.dot_general(r, o3, dnt, preferred_element_type=jnp.float32)
         + lax.dot_general(nb, g_n, dnt, preferred_element_type=jnp.float32)
         + c)
    x = x - jnp.max(x, axis=-1, keepdims=True)
    e = jnp.exp(x)
    out_ref[...] = e / jnp.sum(e, axis=-1, keepdims=True)


@functools.cache
def _make_sc_scatter(n, d, e, nc, ns, c):
    nw = nc * ns
    epw = e // nw
    nchunk = epw // c
    npc = 25
    nstage = nchunk // npc
    rpt = (n // ns) // 8 * 8
    tail = n - rpt * ns
    mesh = plsc.VectorSubcoreMesh(core_axis_name="c", subcore_axis_name="s")

    @functools.partial(
        pl.kernel,
        out_type=jax.ShapeDtypeStruct((nc, n, d), jnp.float32),
        mesh=mesh,
        scratch_types=[
            pltpu.VMEM((npc, c), jnp.int32),
            pltpu.VMEM((npc, c), jnp.int32),
            pltpu.VMEM((npc, c), jnp.float32),
            pltpu.VMEM((c, d), jnp.float32),
            pltpu.VMEM((c, d), jnp.float32),
            pltpu.VMEM_SHARED((n, d), jnp.float32),
            pltpu.SemaphoreType.DMA,
            pltpu.SemaphoreType.DMA,
        ],
    )
    def sc_scatter(i_hbm, src_hbm, dst_hbm, w_hbm, z_hbm, out_hbm,
                   src_v, dst_v, w_v, rows0_v, rows1_v, acc, sem0, sem1):
        ci = lax.axis_index("c")
        si = lax.axis_index("s")
        wid = si * nc + ci
        r0 = pl.multiple_of(si * rpt, 8)
        pltpu.sync_copy(z_hbm.at[pl.ds(r0, rpt)], acc.at[pl.ds(r0, rpt)])
        if tail:
            @pl.when(si == ns - 1)
            def _():
                pltpu.sync_copy(z_hbm.at[pl.ds(rpt * ns, tail)],
                                acc.at[pl.ds(rpt * ns, tail)])
        plsc.subcore_barrier()

        def stage(st, carry):
            pltpu.sync_copy(src_hbm.at[wid, st], src_v)
            pltpu.sync_copy(dst_hbm.at[wid, st], dst_v)
            pltpu.sync_copy(w_hbm.at[wid, st], w_v)
            pltpu.async_copy(i_hbm.at[src_v.at[0]], rows0_v, sem0)

            def body(j, cur, csem, nxt, nsem):
                pltpu.make_async_copy(i_hbm.at[src_v.at[j]], cur, csem).wait()

                @pl.when(j < npc - 1)
                def _():
                    pltpu.async_copy(i_hbm.at[src_v.at[j + 1]], nxt, nsem)

                for g in range(c // 16):
                    wv = w_v[j, pl.ds(g * 16, 16)]
                    for e in range(16):
                        w = wv[e]
                        ei = g * 16 + e
                        for k in range(d // 16):
                            sl = pl.ds(k * 16, 16)
                            cur[ei, sl] = cur[ei, sl] * w
                pltpu.sync_copy(cur, acc.at[dst_v.at[j]], add=True)

            def chunk(j, c2):
                @pl.when(j % 2 == 0)
                def _():
                    body(j, rows0_v, sem0, rows1_v, sem1)

                @pl.when(j % 2 == 1)
                def _():
                    body(j, rows1_v, sem1, rows0_v, sem0)

                return c2

            lax.fori_loop(0, npc, chunk, 0)
            return carry

        lax.fori_loop(0, nstage, stage, 0)
        plsc.subcore_barrier()
        pltpu.sync_copy(acc.at[pl.ds(r0, rpt)], out_hbm.at[ci, pl.ds(r0, rpt)])
        if tail:
            @pl.when(si == ns - 1)
            def _():
                pltpu.sync_copy(acc.at[pl.ds(rpt * ns, tail)],
                                out_hbm.at[ci, pl.ds(rpt * ns, tail)])

    return sc_scatter


def _sc_partials(i_arr, src3, dst3, w3, zeros, nc, ns, c):
    n, d = i_arr.shape
    e = src3.size
    return _make_sc_scatter(n, d, e, nc, ns, c)(i_arr, src3, dst3, w3, zeros)


def kernel(feature, edge_index, edge_weight, W_s, b_s, W_i, b_i, W_r, b_r,
           bn_gamma, bn_beta, toI_W, toI_b, toR_W, toR_b, out_W, out_b):
    n, d = feature.shape
    e = edge_weight.shape[0]
    blk = 1000 if n % 1000 == 0 else n
    nblk = n // blk
    nf = float(n)

    bs = b_s.reshape(1, d)
    bi = b_i.reshape(1, d)
    br = b_r.reshape(1, d)
    gam = bn_gamma.reshape(1, d)
    bet = bn_beta.reshape(1, d)

    row = lambda i: pl.BlockSpec((blk, d), lambda b: (b, 0))
    full = lambda s: pl.BlockSpec(s, lambda b: (0,) * len(s))

    z_s, z_i, z_r, stats = pl.pallas_call(
        _k1_body,
        grid=(nblk,),
        in_specs=[row(0)] + [full((d, d))] * 3 + [full((1, d))] * 3,
        out_specs=[row(0), row(0), row(0), full((8, d))],
        out_shape=[jax.ShapeDtypeStruct((n, d), jnp.float32)] * 3
        + [jax.ShapeDtypeStruct((8, d), jnp.float32)],
    )(feature, W_s, W_i, W_r, bs, bi, br)

    i_arr = pl.pallas_call(
        functools.partial(_k2_body, nf),
        grid=(nblk,),
        in_specs=[row(0), full((8, d)), full((1, d)), full((1, d))],
        out_specs=row(0),
        out_shape=jax.ShapeDtypeStruct((n, d), jnp.float32),
    )(z_i, stats, gam, bet)

    info = plsc.get_sparse_core_info()
    nc, ns = info.num_cores, info.num_subcores
    c = 80
    npc = 25
    src3 = edge_index[0].reshape(nc * ns, -1, npc, c)
    dst3 = edge_index[1].reshape(nc * ns, -1, npc, c)
    w3 = edge_weight.reshape(nc * ns, -1, npc, c)
    zeros = jnp.zeros((n, d), jnp.float32)
    partials = _sc_partials(i_arr, src3, dst3, w3, zeros, nc, ns, c)

    a_m = toI_W[:, :d]
    b_m = toI_W[:, d:]
    o1 = out_W[:, :d]
    o2 = out_W[:, d:2 * d]
    o3 = out_W[:, 2 * d:]

    out = pl.pallas_call(
        functools.partial(_k3_body, nf),
        grid=(nblk,),
        in_specs=[row(0)] * 5 + [full((8, d)), full((1, d)), full((1, d))]
        + [full((d, d))] * 3 + [full((3, d))] * 3
        + [full((1, d)), full((1, d)), full((1, 3))],
        out_specs=pl.BlockSpec((blk, 3), lambda b: (b, 0)),
        out_shape=jax.ShapeDtypeStruct((n, 3), jnp.float32),
    )(z_s, z_r, i_arr, partials[0], partials[1], stats, gam, bet,
      a_m, b_m, toR_W, o1, o2, o3,
      toI_b.reshape(1, d), toR_b.reshape(1, d), out_b.reshape(1, 3))
    return out
